# Initial kernel scaffold; baseline (speedup 1.0000x reference)
import jax
import jax.numpy as jnp
from jax.experimental import pallas as pl


def kernel(X, edge_index, W, b):
    raise NotImplementedError("write your pallas kernel here")


# speedup vs baseline: 5.5069x; 5.5069x over previous
import functools

import jax
import jax.numpy as jnp
import numpy as np
from jax import lax
from jax.experimental import pallas as pl
from jax.experimental.pallas import tpu as pltpu
from jax.experimental.pallas import tpu_sc as plsc

NTILES = 16
NCORES = 2
CHUNK = 128


def _sc_aggregate(x0, x1, src_r, dst_r, zrows, *, n_pad, dh, nch):
    mesh = plsc.VectorSubcoreMesh(core_axis_name="c", subcore_axis_name="s")
    rows_per_tile = n_pad // NTILES

    @functools.partial(
        pl.kernel,
        out_type=jax.ShapeDtypeStruct((NCORES, n_pad, dh), jnp.float32),
        mesh=mesh,
        scratch_types=[
            pltpu.VMEM((nch, CHUNK), jnp.int32),
            pltpu.VMEM((nch, CHUNK), jnp.int32),
            pltpu.VMEM((CHUNK, dh), jnp.float32),
            pltpu.VMEM_SHARED((n_pad, dh), jnp.float32),
            pltpu.SemaphoreType.DMA,
        ],
    )
    def sc_kernel(x0h, x1h, srch, dsth, zh, out, src_v, dst_v, rows_v,
                  agg, sem):
        c = lax.axis_index("c")
        s = lax.axis_index("s")

        pltpu.sync_copy(zh, agg.at[pl.ds(s * rows_per_tile, rows_per_tile)])

        pltpu.sync_copy(srch.at[s], src_v)
        pltpu.sync_copy(dsth.at[s], dst_v)
        plsc.subcore_barrier()

        def make_step(xh):
            def step(j, carry):
                pltpu.async_copy(xh.at[src_v.at[j]], rows_v, sem).wait()
                pltpu.sync_copy(rows_v, agg.at[dst_v.at[j]], add=True)
                return carry
            return step

        @pl.when(c == 0)
        def _():
            lax.fori_loop(0, nch, make_step(x0h), 0)

        @pl.when(c == 1)
        def _():
            lax.fori_loop(0, nch, make_step(x1h), 0)

        plsc.subcore_barrier()

        pltpu.sync_copy(
            agg.at[pl.ds(s * rows_per_tile, rows_per_tile)],
            out.at[c, pl.ds(s * rows_per_tile, rows_per_tile)])

    return sc_kernel(x0, x1, src_r, dst_r, zrows)


def _tc_linear(a0, a1, wt, b2, *, n_pad, dh, d_out, bm):
    def body(a0_ref, a1_ref, wt_ref, b_ref, o_ref):
        acc = jnp.dot(a0_ref[...], wt_ref[0:dh, :],
                      preferred_element_type=jnp.float32)
        acc = acc + jnp.dot(a1_ref[...], wt_ref[dh:, :],
                            preferred_element_type=jnp.float32)
        o_ref[...] = acc + b_ref[...]

    return pl.pallas_call(
        body,
        grid=(n_pad // bm,),
        in_specs=[
            pl.BlockSpec((bm, dh), lambda i: (i, 0)),
            pl.BlockSpec((bm, dh), lambda i: (i, 0)),
            pl.BlockSpec((2 * dh, d_out), lambda i: (0, 0)),
            pl.BlockSpec((1, d_out), lambda i: (0, 0)),
        ],
        out_specs=pl.BlockSpec((bm, d_out), lambda i: (i, 0)),
        out_shape=jax.ShapeDtypeStruct((n_pad, d_out), jnp.float32),
    )(a0, a1, wt, b2)


def kernel(X, edge_index, W, b):
    n, d_in = X.shape
    d_out = W.shape[0]
    e = edge_index.shape[1]
    dh = d_in // 2

    n_pad = ((n + NTILES * CHUNK - 1) // (NTILES * CHUNK)) * (NTILES * CHUNK)
    nch = (e + NTILES * CHUNK - 1) // (NTILES * CHUNK)
    e_pad = NTILES * CHUNK * nch
    npad_e = e_pad - e

    src = edge_index[0]
    dst = edge_index[1]
    if npad_e:
        pad_src = ((np.arange(npad_e) * 37) % n).astype(np.int32)
        pad_dst = (n + (np.arange(npad_e) % (n_pad - n))).astype(np.int32)
        src = jnp.concatenate([src, jnp.asarray(pad_src)])
        dst = jnp.concatenate([dst, jnp.asarray(pad_dst)])
    src_r = src.reshape(NTILES, nch, CHUNK)
    dst_r = dst.reshape(NTILES, nch, CHUNK)

    x0 = X[:, :dh]
    x1 = X[:, dh:]
    zrows = jnp.zeros((n_pad // NTILES, dh), jnp.float32)

    agg = _sc_aggregate(x0, x1, src_r, dst_r, zrows,
                        n_pad=n_pad, dh=dh, nch=nch)

    out = _tc_linear(agg[0], agg[1], W.T, b.reshape(1, d_out),
                     n_pad=n_pad, dh=dh, d_out=d_out, bm=1024)
    return out[:n]


# speedup vs baseline: 8.6667x; 1.5738x over previous
import functools

import jax
import jax.numpy as jnp
import numpy as np
from jax import lax
from jax.experimental import pallas as pl
from jax.experimental.pallas import tpu as pltpu
from jax.experimental.pallas import tpu_sc as plsc

NTILES = 16
NCORES = 2
CHUNK = 128


def _tc_matmul(x, w, *, n, d_in, d_out, dh, bm):
    def body(x_ref, w_ref, y0_ref, y1_ref):
        y = lax.dot_general(x_ref[...], w_ref[...],
                            (((1,), (1,)), ((), ())),
                            preferred_element_type=jnp.float32)
        y0_ref[...] = y[:, :dh]
        y1_ref[...] = y[:, dh:]

    return pl.pallas_call(
        body,
        grid=(n // bm,),
        in_specs=[
            pl.BlockSpec((bm, d_in), lambda i: (i, 0)),
            pl.BlockSpec((d_out, d_in), lambda i: (0, 0)),
        ],
        out_specs=[
            pl.BlockSpec((bm, dh), lambda i: (i, 0)),
            pl.BlockSpec((bm, dh), lambda i: (i, 0)),
        ],
        out_shape=[
            jax.ShapeDtypeStruct((n, dh), jnp.float32),
            jax.ShapeDtypeStruct((n, dh), jnp.float32),
        ],
    )(x, w)


def _sc_aggregate(y0, y1, src_r, dst_r, binit, *, n_pad, dh, nch):
    mesh = plsc.VectorSubcoreMesh(core_axis_name="c", subcore_axis_name="s")
    rpt = n_pad // NTILES

    @functools.partial(
        pl.kernel,
        out_type=jax.ShapeDtypeStruct((n_pad, NCORES * dh), jnp.float32),
        mesh=mesh,
        scratch_types=[
            pltpu.VMEM((4, CHUNK), jnp.int32),
            pltpu.VMEM((CHUNK, dh), jnp.float32),
            pltpu.VMEM((CHUNK, dh), jnp.float32),
            pltpu.VMEM_SHARED((n_pad, dh), jnp.float32),
            pltpu.SemaphoreType.DMA,
            pltpu.SemaphoreType.DMA,
            pltpu.SemaphoreType.DMA,
            pltpu.SemaphoreType.DMA,
            pltpu.SemaphoreType.DMA,
            pltpu.SemaphoreType.DMA,
        ],
    )
    def sc_kernel(y0h, y1h, srch, dsth, bh, out, idx, rows_a, rows_b,
                  agg, si_a, si_b, sd_a, sd_b, sg_a, sg_b):
        c = lax.axis_index("c")
        s = lax.axis_index("s")

        pltpu.sync_copy(bh.at[c], agg.at[pl.ds(s * rpt, rpt)])
        plsc.subcore_barrier()

        def run(yh):
            pltpu.async_copy(srch.at[s, 0], idx.at[0], si_a)
            pltpu.async_copy(srch.at[s, 1], idx.at[1], si_b)
            pltpu.async_copy(dsth.at[s, 0], idx.at[2], sd_a)
            pltpu.async_copy(dsth.at[s, 1], idx.at[3], sd_b)
            pltpu.make_async_copy(srch.at[s, 0], idx.at[0], si_a).wait()
            pltpu.async_copy(yh.at[idx.at[0]], rows_a, sg_a)

            def step(k, carry):
                g = 2 * k
                pltpu.make_async_copy(srch.at[s, g + 1], idx.at[1], si_b).wait()
                pltpu.async_copy(yh.at[idx.at[1]], rows_b, sg_b)
                pltpu.make_async_copy(yh.at[idx.at[0]], rows_a, sg_a).wait()

                @pl.when(g + 2 < nch)
                def _():
                    pltpu.async_copy(srch.at[s, g + 2], idx.at[0], si_a)

                pltpu.make_async_copy(dsth.at[s, g], idx.at[2], sd_a).wait()
                pltpu.sync_copy(rows_a, agg.at[idx.at[2]], add=True)

                @pl.when(g + 2 < nch)
                def _():
                    pltpu.async_copy(dsth.at[s, g + 2], idx.at[2], sd_a)
                    pltpu.make_async_copy(
                        srch.at[s, g + 2], idx.at[0], si_a).wait()
                    pltpu.async_copy(yh.at[idx.at[0]], rows_a, sg_a)

                pltpu.make_async_copy(yh.at[idx.at[1]], rows_b, sg_b).wait()

                @pl.when(g + 3 < nch)
                def _():
                    pltpu.async_copy(srch.at[s, g + 3], idx.at[1], si_b)

                pltpu.make_async_copy(dsth.at[s, g + 1], idx.at[3], sd_b).wait()
                pltpu.sync_copy(rows_b, agg.at[idx.at[3]], add=True)

                @pl.when(g + 3 < nch)
                def _():
                    pltpu.async_copy(dsth.at[s, g + 3], idx.at[3], sd_b)

                return carry

            lax.fori_loop(0, nch // 2, step, 0)

        @pl.when(c == 0)
        def _():
            run(y0h)

        @pl.when(c == 1)
        def _():
            run(y1h)

        plsc.subcore_barrier()

        pltpu.sync_copy(
            agg.at[pl.ds(s * rpt, rpt)],
            out.at[pl.ds(s * rpt, rpt), pl.ds(c * dh, dh)])

    return sc_kernel(y0, y1, src_r, dst_r, binit)


def kernel(X, edge_index, W, b):
    n, d_in = X.shape
    d_out = W.shape[0]
    e = edge_index.shape[1]
    dh = d_out // 2

    n_pad = ((n + 8 * NTILES - 1) // (8 * NTILES)) * (8 * NTILES)
    if n_pad == n:
        n_pad = n + 8 * NTILES
    nch = (e + NTILES * CHUNK - 1) // (NTILES * CHUNK)
    nch += nch % 2
    e_pad = NTILES * CHUNK * nch
    npad_e = e_pad - e

    src = edge_index[0]
    dst = edge_index[1]
    if npad_e:
        pad_src = ((np.arange(npad_e) * 37) % n).astype(np.int32)
        pad_dst = (n + (np.arange(npad_e) % (n_pad - n))).astype(np.int32)
        src = jnp.concatenate([src, jnp.asarray(pad_src)])
        dst = jnp.concatenate([dst, jnp.asarray(pad_dst)])
    src_r = src.reshape(NTILES, nch, CHUNK)
    dst_r = dst.reshape(NTILES, nch, CHUNK)

    y0, y1 = _tc_matmul(X, W, n=n, d_in=d_in, d_out=d_out, dh=dh, bm=1000)

    rpt = n_pad // NTILES
    binit = jnp.broadcast_to(b.reshape(NCORES, 1, dh), (NCORES, rpt, dh))

    out = _sc_aggregate(y0, y1, src_r, dst_r, binit,
                        n_pad=n_pad, dh=dh, nch=nch)
    return out[:n]


# speedup vs baseline: 9.4689x; 1.0926x over previous
import functools

import jax
import jax.numpy as jnp
import numpy as np
from jax import lax
from jax.experimental import pallas as pl
from jax.experimental.pallas import tpu as pltpu
from jax.experimental.pallas import tpu_sc as plsc

NTILES = 16
NCORES = 2
CHUNK = 128


def _tc_matmul(x, w, *, n, d_in, d_out, dh, bm):
    def body(x_ref, w_ref, y0_ref, y1_ref):
        y = lax.dot_general(x_ref[...], w_ref[...],
                            (((1,), (1,)), ((), ())),
                            preferred_element_type=jnp.float32)
        y0_ref[...] = y[:, :dh]
        y1_ref[...] = y[:, dh:]

    return pl.pallas_call(
        body,
        grid=(n // bm,),
        in_specs=[
            pl.BlockSpec((bm, d_in), lambda i: (i, 0)),
            pl.BlockSpec((d_out, d_in), lambda i: (0, 0)),
        ],
        out_specs=[
            pl.BlockSpec((bm, dh), lambda i: (i, 0)),
            pl.BlockSpec((bm, dh), lambda i: (i, 0)),
        ],
        out_shape=[
            jax.ShapeDtypeStruct((n, dh), jnp.float32),
            jax.ShapeDtypeStruct((n, dh), jnp.float32),
        ],
    )(x, w)


def _sc_aggregate(y0, y1, eidx, binit, *, n, n_pad, dh, nch):
    mesh = plsc.VectorSubcoreMesh(core_axis_name="c", subcore_axis_name="s")
    rpt = n_pad // NTILES
    tail = n - (NTILES - 1) * rpt

    @functools.partial(
        pl.kernel,
        out_type=jax.ShapeDtypeStruct((n, NCORES * dh), jnp.float32),
        mesh=mesh,
        scratch_types=[
            pltpu.VMEM((4, CHUNK), jnp.int32),
            pltpu.VMEM((CHUNK, dh), jnp.float32),
            pltpu.VMEM((CHUNK, dh), jnp.float32),
            pltpu.VMEM_SHARED((n_pad, dh), jnp.float32),
            pltpu.SemaphoreType.DMA,
            pltpu.SemaphoreType.DMA,
            pltpu.SemaphoreType.DMA,
            pltpu.SemaphoreType.DMA,
            pltpu.SemaphoreType.DMA,
            pltpu.SemaphoreType.DMA,
        ],
    )
    def sc_kernel(y0h, y1h, eh, bh, out, idx, rows_a, rows_b,
                  agg, si_a, si_b, sd_a, sd_b, sg_a, sg_b):
        c = lax.axis_index("c")
        s = lax.axis_index("s")
        srch = eh.at[0]
        dsth = eh.at[1]

        pltpu.sync_copy(bh.at[c], agg.at[pl.ds(s * rpt, rpt)])
        plsc.subcore_barrier()

        def run(yh):
            pltpu.async_copy(srch.at[s, 0], idx.at[0], si_a)
            pltpu.async_copy(srch.at[s, 1], idx.at[1], si_b)
            pltpu.async_copy(dsth.at[s, 0], idx.at[2], sd_a)
            pltpu.async_copy(dsth.at[s, 1], idx.at[3], sd_b)
            pltpu.make_async_copy(srch.at[s, 0], idx.at[0], si_a).wait()
            pltpu.async_copy(yh.at[idx.at[0]], rows_a, sg_a)

            def step(k, carry):
                g = 2 * k
                pltpu.make_async_copy(srch.at[s, g + 1], idx.at[1], si_b).wait()
                pltpu.async_copy(yh.at[idx.at[1]], rows_b, sg_b)
                pltpu.make_async_copy(yh.at[idx.at[0]], rows_a, sg_a).wait()

                @pl.when(g + 2 < nch)
                def _():
                    pltpu.async_copy(srch.at[s, g + 2], idx.at[0], si_a)

                pltpu.make_async_copy(dsth.at[s, g], idx.at[2], sd_a).wait()
                pltpu.sync_copy(rows_a, agg.at[idx.at[2]], add=True)

                @pl.when(g + 2 < nch)
                def _():
                    pltpu.async_copy(dsth.at[s, g + 2], idx.at[2], sd_a)
                    pltpu.make_async_copy(
                        srch.at[s, g + 2], idx.at[0], si_a).wait()
                    pltpu.async_copy(yh.at[idx.at[0]], rows_a, sg_a)

                pltpu.make_async_copy(yh.at[idx.at[1]], rows_b, sg_b).wait()

                @pl.when(g + 3 < nch)
                def _():
                    pltpu.async_copy(srch.at[s, g + 3], idx.at[1], si_b)

                pltpu.make_async_copy(dsth.at[s, g + 1], idx.at[3], sd_b).wait()
                pltpu.sync_copy(rows_b, agg.at[idx.at[3]], add=True)

                @pl.when(g + 3 < nch)
                def _():
                    pltpu.async_copy(dsth.at[s, g + 3], idx.at[3], sd_b)

                return carry

            lax.fori_loop(0, nch // 2, step, 0)

        @pl.when(c == 0)
        def _():
            run(y0h)

        @pl.when(c == 1)
        def _():
            run(y1h)

        plsc.subcore_barrier()

        @pl.when(s < NTILES - 1)
        def _():
            pltpu.sync_copy(
                agg.at[pl.ds(s * rpt, rpt)],
                out.at[pl.ds(s * rpt, rpt), pl.ds(c * dh, dh)])

        @pl.when(s == NTILES - 1)
        def _():
            pltpu.sync_copy(
                agg.at[pl.ds((NTILES - 1) * rpt, tail)],
                out.at[pl.ds((NTILES - 1) * rpt, tail), pl.ds(c * dh, dh)])

    return sc_kernel(y0, y1, eidx, binit)


def kernel(X, edge_index, W, b):
    n, d_in = X.shape
    d_out = W.shape[0]
    e = edge_index.shape[1]
    dh = d_out // 2

    n_pad = ((n + 8 * NTILES - 1) // (8 * NTILES)) * (8 * NTILES)
    if n_pad == n:
        n_pad = n + 8 * NTILES
    nch = (e + NTILES * CHUNK - 1) // (NTILES * CHUNK)
    nch += nch % 2
    e_pad = NTILES * CHUNK * nch
    npad_e = e_pad - e

    ei = edge_index
    if npad_e:
        pad_src = ((np.arange(npad_e) * 37) % n).astype(np.int32)
        pad_dst = (n + (np.arange(npad_e) % (n_pad - n))).astype(np.int32)
        ei = jnp.concatenate(
            [ei, jnp.asarray(np.stack([pad_src, pad_dst]))], axis=1)
    eidx = ei.reshape(2, NTILES, nch, CHUNK)

    y0, y1 = _tc_matmul(X, W, n=n, d_in=d_in, d_out=d_out, dh=dh, bm=1000)

    rpt = n_pad // NTILES
    binit = jnp.broadcast_to(b.reshape(NCORES, 1, dh), (NCORES, rpt, dh))

    return _sc_aggregate(y0, y1, eidx, binit,
                         n=n, n_pad=n_pad, dh=dh, nch=nch)


# speedup vs baseline: 10.1114x; 1.0679x over previous
import functools

import jax
import jax.numpy as jnp
import numpy as np
from jax import lax
from jax.experimental import pallas as pl
from jax.experimental.pallas import tpu as pltpu
from jax.experimental.pallas import tpu_sc as plsc

NTILES = 16
NCORES = 2
CHUNK = 128


def _tc_matmul(x, w, *, n, d_in, d_out, dh, bm):
    def body(x_ref, w_ref, y0_ref, y1_ref):
        y = lax.dot_general(x_ref[...], w_ref[...],
                            (((1,), (1,)), ((), ())),
                            preferred_element_type=jnp.float32)
        y0_ref[...] = y[:, :dh]
        y1_ref[...] = y[:, dh:]

    return pl.pallas_call(
        body,
        grid=(n // bm,),
        in_specs=[
            pl.BlockSpec((bm, d_in), lambda i: (i, 0)),
            pl.BlockSpec((d_out, d_in), lambda i: (0, 0)),
        ],
        out_specs=[
            pl.BlockSpec((bm, dh), lambda i: (i, 0)),
            pl.BlockSpec((bm, dh), lambda i: (i, 0)),
        ],
        out_shape=[
            jax.ShapeDtypeStruct((n, dh), jnp.float32),
            jax.ShapeDtypeStruct((n, dh), jnp.float32),
        ],
    )(x, w)


def _sc_aggregate(y0, y1, eidx, binit, *, n, n_pad, dh, nch):
    mesh = plsc.VectorSubcoreMesh(core_axis_name="c", subcore_axis_name="s")
    rpt = n_pad // NTILES
    tail = n - (NTILES - 1) * rpt

    @functools.partial(
        pl.kernel,
        out_type=jax.ShapeDtypeStruct((n, NCORES * dh), jnp.float32),
        mesh=mesh,
        scratch_types=[
            pltpu.VMEM((4, CHUNK), jnp.int32),
            pltpu.VMEM((CHUNK, dh), jnp.float32),
            pltpu.VMEM((CHUNK, dh), jnp.float32),
            pltpu.VMEM_SHARED((n_pad, dh), jnp.float32),
            pltpu.SemaphoreType.DMA,
            pltpu.SemaphoreType.DMA,
            pltpu.SemaphoreType.DMA,
            pltpu.SemaphoreType.DMA,
            pltpu.SemaphoreType.DMA,
            pltpu.SemaphoreType.DMA,
        ],
    )
    def sc_kernel(y0h, y1h, eh, bh, out, idx, rows_a, rows_b,
                  agg, si_a, si_b, sd_a, sd_b, sg_a, sg_b):
        c = lax.axis_index("c")
        s = lax.axis_index("s")
        srch = eh.at[0]
        dsth = eh.at[1]

        pltpu.sync_copy(bh.at[c], agg.at[pl.ds(s * rpt, rpt)])
        plsc.subcore_barrier()

        def run(yh):
            pltpu.async_copy(srch.at[s, 0], idx.at[0], si_a)
            pltpu.async_copy(srch.at[s, 1], idx.at[1], si_b)
            pltpu.async_copy(dsth.at[s, 0], idx.at[2], sd_a)
            pltpu.async_copy(dsth.at[s, 1], idx.at[3], sd_b)
            pltpu.make_async_copy(srch.at[s, 0], idx.at[0], si_a).wait()
            pltpu.async_copy(yh.at[idx.at[0]], rows_a, sg_a)

            def step(k, carry):
                g = 2 * k
                pltpu.make_async_copy(srch.at[s, g + 1], idx.at[1], si_b).wait()
                pltpu.async_copy(yh.at[idx.at[1]], rows_b, sg_b)
                pltpu.make_async_copy(yh.at[idx.at[0]], rows_a, sg_a).wait()

                @pl.when(g + 2 < nch)
                def _():
                    pltpu.async_copy(srch.at[s, g + 2], idx.at[0], si_a)

                pltpu.make_async_copy(dsth.at[s, g], idx.at[2], sd_a).wait()

                @pl.when(g + 2 < nch)
                def _():
                    pltpu.async_copy(dsth.at[s, g + 2], idx.at[2], sd_a)
                    pltpu.make_async_copy(
                        srch.at[s, g + 2], idx.at[0], si_a).wait()
                    pltpu.async_copy(yh.at[idx.at[0]], rows_a, sg_a)

                pltpu.make_async_copy(yh.at[idx.at[1]], rows_b, sg_b).wait()

                @pl.when(g + 3 < nch)
                def _():
                    pltpu.async_copy(srch.at[s, g + 3], idx.at[1], si_b)

                pltpu.make_async_copy(dsth.at[s, g + 1], idx.at[3], sd_b).wait()

                @pl.when(g + 3 < nch)
                def _():
                    pltpu.async_copy(dsth.at[s, g + 3], idx.at[3], sd_b)

                return carry

            lax.fori_loop(0, nch // 2, step, 0)

        @pl.when(c == 0)
        def _():
            run(y0h)

        @pl.when(c == 1)
        def _():
            run(y1h)

        plsc.subcore_barrier()

        @pl.when(s < NTILES - 1)
        def _():
            pltpu.sync_copy(
                agg.at[pl.ds(s * rpt, rpt)],
                out.at[pl.ds(s * rpt, rpt), pl.ds(c * dh, dh)])

        @pl.when(s == NTILES - 1)
        def _():
            pltpu.sync_copy(
                agg.at[pl.ds((NTILES - 1) * rpt, tail)],
                out.at[pl.ds((NTILES - 1) * rpt, tail), pl.ds(c * dh, dh)])

    return sc_kernel(y0, y1, eidx, binit)


def kernel(X, edge_index, W, b):
    n, d_in = X.shape
    d_out = W.shape[0]
    e = edge_index.shape[1]
    dh = d_out // 2

    n_pad = ((n + 8 * NTILES - 1) // (8 * NTILES)) * (8 * NTILES)
    if n_pad == n:
        n_pad = n + 8 * NTILES
    nch = (e + NTILES * CHUNK - 1) // (NTILES * CHUNK)
    nch += nch % 2
    e_pad = NTILES * CHUNK * nch
    npad_e = e_pad - e

    ei = edge_index
    if npad_e:
        pad_src = ((np.arange(npad_e) * 37) % n).astype(np.int32)
        pad_dst = (n + (np.arange(npad_e) % (n_pad - n))).astype(np.int32)
        ei = jnp.concatenate(
            [ei, jnp.asarray(np.stack([pad_src, pad_dst]))], axis=1)
    eidx = ei.reshape(2, NTILES, nch, CHUNK)

    y0, y1 = _tc_matmul(X, W, n=n, d_in=d_in, d_out=d_out, dh=dh, bm=1000)

    rpt = n_pad // NTILES
    binit = jnp.broadcast_to(b.reshape(NCORES, 1, dh), (NCORES, rpt, dh))

    return _sc_aggregate(y0, y1, eidx, binit,
                         n=n, n_pad=n_pad, dh=dh, nch=nch)


# speedup vs baseline: 10.2757x; 1.0162x over previous
import functools

import jax
import jax.numpy as jnp
import numpy as np
from jax import lax
from jax.experimental import pallas as pl
from jax.experimental.pallas import tpu as pltpu
from jax.experimental.pallas import tpu_sc as plsc

NTILES = 16
NCORES = 2
CHUNK = 112
NSLOT = 3


def _tc_matmul(x, w, *, n, d_in, d_out, dh, bm):
    def body(x_ref, w_ref, y0_ref, y1_ref):
        y = lax.dot_general(x_ref[...], w_ref[...],
                            (((1,), (1,)), ((), ())),
                            preferred_element_type=jnp.float32)
        y0_ref[...] = y[:, :dh]
        y1_ref[...] = y[:, dh:]

    return pl.pallas_call(
        body,
        grid=(n // bm,),
        in_specs=[
            pl.BlockSpec((bm, d_in), lambda i: (i, 0)),
            pl.BlockSpec((d_out, d_in), lambda i: (0, 0)),
        ],
        out_specs=[
            pl.BlockSpec((bm, dh), lambda i: (i, 0)),
            pl.BlockSpec((bm, dh), lambda i: (i, 0)),
        ],
        out_shape=[
            jax.ShapeDtypeStruct((n, dh), jnp.float32),
            jax.ShapeDtypeStruct((n, dh), jnp.float32),
        ],
    )(x, w)


def _sc_aggregate(y0, y1, eidx, binit, *, n, n_pad, dh, nch):
    mesh = plsc.VectorSubcoreMesh(core_axis_name="c", subcore_axis_name="s")
    rpt = n_pad // NTILES
    tail = n - (NTILES - 1) * rpt

    @functools.partial(
        pl.kernel,
        out_type=jax.ShapeDtypeStruct((n, NCORES * dh), jnp.float32),
        mesh=mesh,
        scratch_types=[
            pltpu.VMEM((NSLOT, CHUNK), jnp.int32),
            pltpu.VMEM((NSLOT, CHUNK), jnp.int32),
            pltpu.VMEM((NSLOT, CHUNK, dh), jnp.float32),
            pltpu.VMEM_SHARED((n_pad, dh), jnp.float32),
            [pltpu.SemaphoreType.DMA] * NSLOT,
            [pltpu.SemaphoreType.DMA] * NSLOT,
            [pltpu.SemaphoreType.DMA] * NSLOT,
        ],
    )
    def sc_kernel(y0h, y1h, eh, bh, out, srcx, dstx, rows,
                  agg, si, sd, sg):
        c = lax.axis_index("c")
        s = lax.axis_index("s")
        srch = eh.at[0]
        dsth = eh.at[1]

        pltpu.sync_copy(bh.at[c], agg.at[pl.ds(s * rpt, rpt)])
        plsc.subcore_barrier()

        def run(yh):
            for p in range(NSLOT):
                pltpu.async_copy(srch.at[s, p], srcx.at[p], si[p])
                pltpu.async_copy(dsth.at[s, p], dstx.at[p], sd[p])
            for p in range(NSLOT):
                pltpu.make_async_copy(srch.at[s, p], srcx.at[p], si[p]).wait()
                pltpu.async_copy(yh.at[srcx.at[p]], rows.at[p], sg[p])

            def step(k, carry):
                j0 = NSLOT * k
                for p in range(NSLOT):
                    j = j0 + p
                    pltpu.make_async_copy(
                        yh.at[srcx.at[p]], rows.at[p], sg[p]).wait()

                    @pl.when(j + NSLOT < nch)
                    def _():
                        pltpu.async_copy(
                            srch.at[s, j + NSLOT], srcx.at[p], si[p])

                    pltpu.make_async_copy(
                        dsth.at[s, j], dstx.at[p], sd[p]).wait()
                    pltpu.sync_copy(rows.at[p], agg.at[dstx.at[p]], add=True)

                    @pl.when(j + NSLOT < nch)
                    def _():
                        pltpu.async_copy(
                            dsth.at[s, j + NSLOT], dstx.at[p], sd[p])
                        pltpu.make_async_copy(
                            srch.at[s, j + NSLOT], srcx.at[p], si[p]).wait()
                        pltpu.async_copy(yh.at[srcx.at[p]], rows.at[p], sg[p])

                return carry

            lax.fori_loop(0, nch // NSLOT, step, 0)

        @pl.when(c == 0)
        def _():
            run(y0h)

        @pl.when(c == 1)
        def _():
            run(y1h)

        plsc.subcore_barrier()

        @pl.when(s < NTILES - 1)
        def _():
            pltpu.sync_copy(
                agg.at[pl.ds(s * rpt, rpt)],
                out.at[pl.ds(s * rpt, rpt), pl.ds(c * dh, dh)])

        @pl.when(s == NTILES - 1)
        def _():
            pltpu.sync_copy(
                agg.at[pl.ds((NTILES - 1) * rpt, tail)],
                out.at[pl.ds((NTILES - 1) * rpt, tail), pl.ds(c * dh, dh)])

    return sc_kernel(y0, y1, eidx, binit)


def kernel(X, edge_index, W, b):
    n, d_in = X.shape
    d_out = W.shape[0]
    e = edge_index.shape[1]
    dh = d_out // 2

    n_pad = ((n + 8 * NTILES - 1) // (8 * NTILES)) * (8 * NTILES)
    if n_pad == n:
        n_pad = n + 8 * NTILES
    nch = (e + NTILES * CHUNK - 1) // (NTILES * CHUNK)
    nch = ((nch + NSLOT - 1) // NSLOT) * NSLOT
    e_pad = NTILES * CHUNK * nch
    npad_e = e_pad - e

    ei = edge_index
    if npad_e:
        pad_src = ((np.arange(npad_e) * 37) % n).astype(np.int32)
        pad_dst = (n + (np.arange(npad_e) % (n_pad - n))).astype(np.int32)
        ei = jnp.concatenate(
            [ei, jnp.asarray(np.stack([pad_src, pad_dst]))], axis=1)
    eidx = ei.reshape(2, NTILES, nch, CHUNK)

    y0, y1 = _tc_matmul(X, W, n=n, d_in=d_in, d_out=d_out, dh=dh, bm=1000)

    rpt = n_pad // NTILES
    binit = jnp.broadcast_to(b.reshape(NCORES, 1, dh), (NCORES, rpt, dh))

    return _sc_aggregate(y0, y1, eidx, binit,
                         n=n, n_pad=n_pad, dh=dh, nch=nch)
